# baseline (speedup 1.0000x reference)
import functools

import jax
import jax.numpy as jnp
from jax import lax
from jax.experimental import pallas as pl
from jax.experimental.pallas import tpu as pltpu
from jax.experimental.pallas import tpu_sc as plsc

_D = 26
_K = 64


def _sc_gather(v3d, fi):
    mesh = plsc.ScalarSubcoreMesh(axis_name="c", num_cores=1)

    @functools.partial(
        pl.kernel,
        mesh=mesh,
        out_type=jax.ShapeDtypeStruct((_D, _K), jnp.float32),
        scratch_types=[
            pltpu.SMEM((_D,), jnp.int32),
            pltpu.SemaphoreType.DMA,
        ],
    )
    def k(table_hbm, fi_hbm, out_hbm, fi_s, sem):
        cid = lax.axis_index("c")

        @pl.when(cid == 0)
        def _():
            pltpu.sync_copy(fi_hbm, fi_s)
            copies = []
            for i in range(_D):
                copies.append(
                    pltpu.async_copy(
                        table_hbm.at[fi_s[i], i], out_hbm.at[i], sem))
            for c in copies:
                c.wait()

    return k(v3d, fi)


def _tc_body(xt_ref, vm_ref, wt_ref, b_ref, o_ref):
    xt = xt_ref[...]
    vm = vm_ref[...]
    s = lax.dot_general(vm, vm, (((1,), (1,)), ((), ())),
                        preferred_element_type=jnp.float32)
    ii = lax.broadcasted_iota(jnp.int32, (_D, _D), 0)
    jj = lax.broadcasted_iota(jnp.int32, (_D, _D), 1)
    m = jnp.where(jj > ii, s, jnp.float32(0.0))
    t = lax.dot_general(m, xt, (((1,), (0,)), ((), ())),
                        preferred_element_type=jnp.float32)
    u = (t + wt_ref[...]) * xt
    z = jnp.sum(u, axis=0, keepdims=True) + b_ref[0, 0]
    o_ref[...] = (1.0 / (1.0 + jnp.exp(-z)))[None]


def kernel(x, field_indices, W, b, v):
    batch = x.shape[0]
    vmat = _sc_gather(v, field_indices.astype(jnp.int32))

    nblk = 8
    bblk = batch // nblk
    out = pl.pallas_call(
        _tc_body,
        grid=(nblk,),
        in_specs=[
            pl.BlockSpec((_D, bblk), lambda i: (0, i)),
            pl.BlockSpec((_D, _K), lambda i: (0, 0)),
            pl.BlockSpec((_D, 1), lambda i: (0, 0)),
            pl.BlockSpec((1, 1), lambda i: (0, 0)),
        ],
        out_specs=pl.BlockSpec((1, 1, bblk), lambda i: (i, 0, 0)),
        out_shape=jax.ShapeDtypeStruct((nblk, 1, bblk), jnp.float32),
    )(x.T, vmat, W.reshape(_D, 1), b.reshape(1, 1))
    return out.reshape(batch, 1)


# speedup vs baseline: 1.1210x; 1.1210x over previous
import functools

import jax
import jax.numpy as jnp
from jax import lax
from jax.experimental import pallas as pl
from jax.experimental.pallas import tpu as pltpu
from jax.experimental.pallas import tpu_sc as plsc

_D = 26
_K = 64


def _sc_gather(v3d, fi):
    mesh = plsc.ScalarSubcoreMesh(axis_name="c", num_cores=1)

    @functools.partial(
        pl.kernel,
        mesh=mesh,
        out_type=jax.ShapeDtypeStruct((_D, _K), jnp.float32),
        scratch_types=[
            pltpu.SMEM((_D,), jnp.int32),
            pltpu.SemaphoreType.DMA,
        ],
    )
    def k(table_hbm, fi_hbm, out_hbm, fi_s, sem):
        cid = lax.axis_index("c")

        @pl.when(cid == 0)
        def _():
            pltpu.sync_copy(fi_hbm, fi_s)
            copies = []
            for i in range(_D):
                copies.append(
                    pltpu.async_copy(
                        table_hbm.at[fi_s[i], i], out_hbm.at[i], sem))
            for c in copies:
                c.wait()

    return k(v3d, fi)


def _tc_body(xt_ref, vm_ref, wt_ref, b_ref, o_ref):
    xt = xt_ref[...]
    vm = vm_ref[...]
    s = lax.dot_general(vm, vm, (((1,), (1,)), ((), ())),
                        preferred_element_type=jnp.float32)
    ii = lax.broadcasted_iota(jnp.int32, (_D, _D), 0)
    jj = lax.broadcasted_iota(jnp.int32, (_D, _D), 1)
    m = jnp.where(jj > ii, s, jnp.float32(0.0))
    t = lax.dot_general(m, xt, (((1,), (0,)), ((), ())),
                        preferred_element_type=jnp.float32)
    u = (t + wt_ref[...]) * xt
    z = jnp.sum(u, axis=0, keepdims=True) + b_ref[0, 0]
    o_ref[...] = (1.0 / (1.0 + jnp.exp(-z)))[None]


def kernel(x, field_indices, W, b, v):
    batch = x.shape[0]
    vmat = _sc_gather(v, field_indices.astype(jnp.int32))

    nblk = 4
    bblk = batch // nblk
    out = pl.pallas_call(
        _tc_body,
        grid=(nblk,),
        in_specs=[
            pl.BlockSpec((_D, bblk), lambda i: (0, i)),
            pl.BlockSpec((_D, _K), lambda i: (0, 0)),
            pl.BlockSpec((_D, 1), lambda i: (0, 0)),
            pl.BlockSpec((1, 1), lambda i: (0, 0)),
        ],
        out_specs=pl.BlockSpec((1, 1, bblk), lambda i: (i, 0, 0)),
        out_shape=jax.ShapeDtypeStruct((nblk, 1, bblk), jnp.float32),
    )(x.T, vmat, W.reshape(_D, 1), b.reshape(1, 1))
    return out.reshape(batch, 1)


# speedup vs baseline: 1.1762x; 1.0492x over previous
import functools

import jax
import jax.numpy as jnp
from jax import lax
from jax.experimental import pallas as pl
from jax.experimental.pallas import tpu as pltpu
from jax.experimental.pallas import tpu_sc as plsc

_D = 26
_K = 64


def _sc_gather(v3d, fi):
    mesh = plsc.ScalarSubcoreMesh(axis_name="c", num_cores=1)

    @functools.partial(
        pl.kernel,
        mesh=mesh,
        out_type=jax.ShapeDtypeStruct((_D, _K), jnp.float32),
        scratch_types=[
            pltpu.SMEM((_D,), jnp.int32),
            pltpu.SemaphoreType.DMA,
        ],
    )
    def k(table_hbm, fi_hbm, out_hbm, fi_s, sem):
        cid = lax.axis_index("c")

        @pl.when(cid == 0)
        def _():
            pltpu.sync_copy(fi_hbm, fi_s)
            copies = []
            for i in range(_D):
                copies.append(
                    pltpu.async_copy(
                        table_hbm.at[fi_s[i], i], out_hbm.at[i], sem))
            for c in copies:
                c.wait()

    return k(v3d, fi)


def _tc_body(xt_ref, vm_ref, wt_ref, b_ref, o_ref):
    xt = xt_ref[...]
    vm = vm_ref[...]
    s = lax.dot_general(vm, vm, (((1,), (1,)), ((), ())),
                        preferred_element_type=jnp.float32)
    ii = lax.broadcasted_iota(jnp.int32, (_D, _D), 0)
    jj = lax.broadcasted_iota(jnp.int32, (_D, _D), 1)
    m = jnp.where(jj > ii, s, jnp.float32(0.0))
    t = lax.dot_general(m, xt, (((1,), (0,)), ((), ())),
                        preferred_element_type=jnp.float32)
    u = (t + wt_ref[...]) * xt
    z = jnp.sum(u, axis=0, keepdims=True) + b_ref[0, 0]
    o_ref[...] = (1.0 / (1.0 + jnp.exp(-z)))[None]


def kernel(x, field_indices, W, b, v):
    batch = x.shape[0]
    vmat = _sc_gather(v, field_indices.astype(jnp.int32))

    nblk = 2
    bblk = batch // nblk
    out = pl.pallas_call(
        _tc_body,
        grid=(nblk,),
        in_specs=[
            pl.BlockSpec((_D, bblk), lambda i: (0, i)),
            pl.BlockSpec((_D, _K), lambda i: (0, 0)),
            pl.BlockSpec((_D, 1), lambda i: (0, 0)),
            pl.BlockSpec((1, 1), lambda i: (0, 0)),
        ],
        out_specs=pl.BlockSpec((1, 1, bblk), lambda i: (i, 0, 0)),
        out_shape=jax.ShapeDtypeStruct((nblk, 1, bblk), jnp.float32),
    )(x.T, vmat, W.reshape(_D, 1), b.reshape(1, 1))
    return out.reshape(batch, 1)
